# baseline (speedup 1.0000x reference)
import functools

import jax
import jax.numpy as jnp
from jax import lax
from jax.experimental import pallas as pl
from jax.experimental.pallas import tpu as pltpu
from jax.experimental.pallas import tpu_sc as plsc

B, L, D = 1024, 200, 64
N = B * L
NC, NS, LANES = 2, 16, 16
NW = NC * NS
TPW = N // NW
GS = 128
NG = TPW // GS

V = 1000000
NBLK = 7813
VPAD = NBLK * 128
BPW = (NBLK + NW - 1) // NW

_MESH = dict(core_axis_name="c", subcore_axis_name="s")
_PARAMS = pltpu.CompilerParams(use_tc_tiling_on_sc=True,
                               needs_layout_passes=False)


def _sc_relayout(tok_t):
    @functools.partial(
        pl.kernel,
        mesh=plsc.VectorSubcoreMesh(**_MESH),
        compiler_params=_PARAMS,
        out_type=jax.ShapeDtypeStruct((VPAD, 2 * D), jnp.float32),
        scratch_types=[
            pltpu.VMEM((D, GS), jnp.float32),
            pltpu.VMEM((GS, 2 * D), jnp.float32),
        ],
    )
    def k1(tok_hbm, t128_hbm, sbuf, res_v):
        wid = lax.axis_index("s") * NC + lax.axis_index("c")

        def zrow(t, c):
            for cc in range(D // LANES):
                res_v[t, pl.ds(D + cc * LANES, LANES)] = jnp.zeros(
                    (LANES,), jnp.float32)
            return c

        lax.fori_loop(0, GS, zrow, 0)

        def block(g, carry):
            blk = wid + NW * g

            @pl.when(blk < NBLK)
            def _():
                src_off = pl.multiple_of(blk * GS, GS)
                pltpu.sync_copy(tok_hbm.at[:, pl.ds(src_off, GS)], sbuf)

                def drow(d, c2):
                    dcol = jnp.zeros((LANES,), jnp.int32) + d
                    for t0 in range(GS // LANES):
                        rows = t0 * LANES + lax.iota(jnp.int32, LANES)
                        vals = sbuf[d, pl.ds(t0 * LANES, LANES)]
                        plsc.store_scatter(res_v, [rows, dcol], vals)
                    return c2

                lax.fori_loop(0, D, drow, 0, unroll=2)
                dst_off = pl.multiple_of(blk * GS, GS)
                pltpu.sync_copy(res_v, t128_hbm.at[pl.ds(dst_off, GS)])

            return carry

        lax.fori_loop(0, BPW, block, 0)

    return k1(tok_t)


def _sc_embed(tok128, tidx2, aidx2, peseg):
    @functools.partial(
        pl.kernel,
        mesh=plsc.VectorSubcoreMesh(**_MESH),
        compiler_params=_PARAMS,
        out_type=jax.ShapeDtypeStruct((N // 2, 2 * D), jnp.float32),
        scratch_types=[
            pltpu.VMEM((TPW,), jnp.int32),
            pltpu.VMEM((TPW,), jnp.int32),
            pltpu.VMEM((GS, 2 * D), jnp.float32),
            pltpu.VMEM((GS, 2 * D), jnp.float32),
            pltpu.VMEM((GS // 2, 2 * D), jnp.float32),
            pltpu.SemaphoreType.DMA,
            pltpu.SemaphoreType.DMA,
        ],
    )
    def k2(tok_hbm, tidx_hbm, aidx_hbm, peseg_hbm, out_hbm,
           tidx_v, aidx_v, tok_v, add_v, res_v, sem_t, sem_a):
        wid = lax.axis_index("s") * NC + lax.axis_index("c")
        pltpu.sync_copy(tidx_hbm.at[wid], tidx_v)
        pltpu.sync_copy(aidx_hbm.at[wid], aidx_v)

        def group(g, carry):
            gbase = g * GS
            cp_t = pltpu.async_copy(tok_hbm.at[tidx_v.at[pl.ds(gbase, GS)]],
                                    tok_v, sem_t)
            cp_a = pltpu.async_copy(peseg_hbm.at[aidx_v.at[pl.ds(gbase, GS)]],
                                    add_v, sem_a)
            cp_t.wait()
            cp_a.wait()

            def pair(rp, c2):
                for half in range(2):
                    r = 2 * rp + half
                    for c in range(D // LANES):
                        src = pl.ds(c * LANES, LANES)
                        dst = pl.ds(half * D + c * LANES, LANES)
                        res_v[rp, dst] = tok_v[r, src] + add_v[r, src]
                return c2

            lax.fori_loop(0, GS // 2, pair, 0, unroll=2)
            off = pl.multiple_of(wid * (TPW // 2) + g * (GS // 2), 8)
            pltpu.sync_copy(res_v, out_hbm.at[pl.ds(off, GS // 2)])
            return carry

        lax.fori_loop(0, NG, group, 0)

    return k2(tok128, tidx2, aidx2, peseg)


def kernel(sequence, segment_labels, tok_table, seg_table, pe):
    tok128 = _sc_relayout(tok_table.T)
    tidx2 = sequence.astype(jnp.int32).reshape(NW, TPW)
    l_pos = jnp.arange(L, dtype=jnp.int32)
    aidx2 = (segment_labels.astype(jnp.int32) * L + l_pos[None, :]).reshape(NW, TPW)
    peseg = (seg_table[:, None, :] + pe[0, :L, :][None, :, :]).reshape(3 * L, D)
    peseg = jnp.concatenate([peseg, peseg], axis=1)
    out = _sc_embed(tok128, tidx2, aidx2, peseg)
    return out.reshape(B, L, D)


# speedup vs baseline: 1.1577x; 1.1577x over previous
import functools

import jax
import jax.numpy as jnp
from jax import lax
from jax.experimental import pallas as pl
from jax.experimental.pallas import tpu as pltpu
from jax.experimental.pallas import tpu_sc as plsc

B, L, D = 1024, 200, 64
N = B * L
NC, NS, LANES = 2, 16, 16
NW = NC * NS
TPW = N // NW
GS = 128
NG = TPW // GS

V = 1000000
NBLK = 7813
VPAD = NBLK * 128
BPW = (NBLK + NW - 1) // NW

_MESH = dict(core_axis_name="c", subcore_axis_name="s")
_PARAMS = pltpu.CompilerParams(use_tc_tiling_on_sc=True,
                               needs_layout_passes=False,
                               disable_bounds_checks=True)


def _sc_relayout(tok_t):
    @functools.partial(
        pl.kernel,
        mesh=plsc.VectorSubcoreMesh(**_MESH),
        compiler_params=_PARAMS,
        out_type=jax.ShapeDtypeStruct((VPAD, 2 * D), jnp.float32),
        scratch_types=[
            pltpu.VMEM((2, D, GS), jnp.float32),
            pltpu.VMEM((GS, 2 * D), jnp.float32),
            [pltpu.SemaphoreType.DMA] * 2,
        ],
    )
    def k1(tok_hbm, t128_hbm, sbuf, res_v, sem_r):
        wid = lax.axis_index("s") * NC + lax.axis_index("c")
        iota16 = lax.iota(jnp.int32, LANES)
        rows_l = [t0 * LANES + iota16 for t0 in range(GS // LANES)]
        zero_i = iota16 * 0
        zero_f = jnp.zeros((LANES,), jnp.float32)

        def zrow(t, c):
            for cc in range(D // LANES):
                res_v[t, pl.ds(D + cc * LANES, LANES)] = zero_f
            return c

        lax.fori_loop(0, GS, zrow, 0)

        def fire(g, b):
            blk = wid + NW * g

            @pl.when(blk < NBLK)
            def _():
                src_off = pl.multiple_of(blk * GS, GS)
                pltpu.async_copy(tok_hbm.at[:, pl.ds(src_off, GS)],
                                 sbuf.at[b], sem_r[b])

        for b in range(2):
            fire(b, b)

        def outer(kk, carry):
            for b in range(2):
                g = 2 * kk + b
                blk = wid + NW * g

                @pl.when(blk < NBLK)
                def _():
                    pltpu.make_async_copy(tok_hbm.at[:, pl.ds(0, GS)],
                                          sbuf.at[b], sem_r[b]).wait()

                    def drow(d, c2):
                        dcol = zero_i + d
                        for t0 in range(GS // LANES):
                            vals = sbuf[b, d, pl.ds(t0 * LANES, LANES)]
                            plsc.store_scatter(res_v, [rows_l[t0], dcol], vals)
                        return c2

                    lax.fori_loop(0, D, drow, 0, unroll=8)
                    dst_off = pl.multiple_of(blk * GS, GS)
                    pltpu.sync_copy(res_v, t128_hbm.at[pl.ds(dst_off, GS)])
                    fire(g + 2, b)

            return carry

        lax.fori_loop(0, (BPW + 1) // 2, outer, 0)

    return k1(tok_t)


def _sc_embed(tok128, tidx2, aidx2, peseg):
    @functools.partial(
        pl.kernel,
        mesh=plsc.VectorSubcoreMesh(**_MESH),
        compiler_params=_PARAMS,
        out_type=jax.ShapeDtypeStruct((N // 2, 2 * D), jnp.float32),
        scratch_types=[
            pltpu.VMEM((TPW,), jnp.int32),
            pltpu.VMEM((TPW,), jnp.int32),
            pltpu.VMEM((GS, 2 * D), jnp.float32),
            pltpu.VMEM((GS, 2 * D), jnp.float32),
            pltpu.VMEM((GS // 2, 2 * D), jnp.float32),
            pltpu.SemaphoreType.DMA,
            pltpu.SemaphoreType.DMA,
        ],
    )
    def k2(tok_hbm, tidx_hbm, aidx_hbm, peseg_hbm, out_hbm,
           tidx_v, aidx_v, tok_v, add_v, res_v, sem_t, sem_a):
        wid = lax.axis_index("s") * NC + lax.axis_index("c")
        pltpu.sync_copy(tidx_hbm.at[wid], tidx_v)
        pltpu.sync_copy(aidx_hbm.at[wid], aidx_v)

        def group(g, carry):
            gbase = g * GS
            cp_t = pltpu.async_copy(tok_hbm.at[tidx_v.at[pl.ds(gbase, GS)]],
                                    tok_v, sem_t)
            cp_a = pltpu.async_copy(peseg_hbm.at[aidx_v.at[pl.ds(gbase, GS)]],
                                    add_v, sem_a)
            cp_t.wait()
            cp_a.wait()

            def pair(rp, c2):
                for half in range(2):
                    r = 2 * rp + half
                    for c in range(D // LANES):
                        src = pl.ds(c * LANES, LANES)
                        dst = pl.ds(half * D + c * LANES, LANES)
                        res_v[rp, dst] = tok_v[r, src] + add_v[r, src]
                return c2

            lax.fori_loop(0, GS // 2, pair, 0, unroll=2)
            off = pl.multiple_of(wid * (TPW // 2) + g * (GS // 2), 8)
            pltpu.sync_copy(res_v, out_hbm.at[pl.ds(off, GS // 2)])
            return carry

        lax.fori_loop(0, NG, group, 0)

    return k2(tok128, tidx2, aidx2, peseg)


def kernel(sequence, segment_labels, tok_table, seg_table, pe):
    tok128 = _sc_relayout(tok_table.T)
    tidx2 = sequence.astype(jnp.int32).reshape(NW, TPW)
    l_pos = jnp.arange(L, dtype=jnp.int32)
    aidx2 = (segment_labels.astype(jnp.int32) * L + l_pos[None, :]).reshape(NW, TPW)
    peseg = (seg_table[:, None, :] + pe[0, :L, :][None, :, :]).reshape(3 * L, D)
    peseg = jnp.concatenate([peseg, peseg], axis=1)
    out = _sc_embed(tok128, tidx2, aidx2, peseg)
    return out.reshape(B, L, D)


# speedup vs baseline: 2.3733x; 2.0499x over previous
import functools

import jax
import jax.numpy as jnp
from jax import lax
from jax.experimental import pallas as pl
from jax.experimental.pallas import tpu as pltpu
from jax.experimental.pallas import tpu_sc as plsc

B, L, D = 1024, 200, 64
N = B * L
NC, NS, LANES = 2, 16, 16
NW = NC * NS
TPW = N // NW
GS = 128
NG = TPW // GS


def _sc_embed(tok_table, tidx3, aidx3, peseg):
    mesh = plsc.VectorSubcoreMesh(core_axis_name="c", subcore_axis_name="s")

    @functools.partial(
        pl.kernel,
        mesh=mesh,
        compiler_params=pltpu.CompilerParams(use_tc_tiling_on_sc=False),
        out_type=jax.ShapeDtypeStruct((N, D), jnp.float32),
        scratch_types=[
            pltpu.VMEM((NG, GS), jnp.int32),
            pltpu.VMEM((NG, GS), jnp.int32),
            pltpu.VMEM((GS, D), jnp.float32),
            pltpu.VMEM((GS, D), jnp.float32),
            pltpu.SemaphoreType.DMA,
            pltpu.SemaphoreType.DMA,
        ],
    )
    def k(tok_hbm, tidx_hbm, aidx_hbm, peseg_hbm, out_hbm,
          tidx_v, aidx_v, tok_v, add_v, sem_t, sem_a):
        wid = lax.axis_index("s") * NC + lax.axis_index("c")
        pltpu.sync_copy(tidx_hbm.at[wid], tidx_v)
        pltpu.sync_copy(aidx_hbm.at[wid], aidx_v)

        def group(g, carry):
            cp_t = pltpu.async_copy(tok_hbm.at[tidx_v.at[g]], tok_v, sem_t)
            cp_a = pltpu.async_copy(peseg_hbm.at[aidx_v.at[g]], add_v, sem_a)
            cp_t.wait()
            cp_a.wait()

            def row(r, c2):
                for c in range(D // LANES):
                    sl = pl.ds(c * LANES, LANES)
                    tok_v[r, sl] = tok_v[r, sl] + add_v[r, sl]
                return c2

            lax.fori_loop(0, GS, row, 0)
            pltpu.sync_copy(tok_v, out_hbm.at[pl.ds(wid * TPW + g * GS, GS)])
            return carry

        lax.fori_loop(0, NG, group, 0)

    return k(tok_table, tidx3, aidx3, peseg)


def kernel(sequence, segment_labels, tok_table, seg_table, pe):
    tidx3 = sequence.astype(jnp.int32).reshape(NW, NG, GS)
    l_pos = jnp.arange(L, dtype=jnp.int32)
    aidx3 = (segment_labels.astype(jnp.int32) * L + l_pos[None, :]).reshape(NW, NG, GS)
    peseg = (seg_table[:, None, :] + pe[0, :L, :][None, :, :]).reshape(3 * L, D)
    out = _sc_embed(tok_table, tidx3, aidx3, peseg)
    return out.reshape(B, L, D)
